# baseline (speedup 1.0000x reference)
import functools
import jax
import jax.numpy as jnp
from jax import lax
from jax.experimental import pallas as pl
from jax.experimental.pallas import tpu as pltpu
from jax.experimental.pallas import tpu_sc as plsc

D_MODEL = 1024
MONTH_ROWS = 13
HOUR_ROWS = 25
COMB_ROWS = MONTH_ROWS * HOUR_ROWS
NC = 2
NS = 16
NW = NC * NS
L = 16

N_TOTAL = 4 * 4096
ROWS_PER_W = N_TOTAL // NW
CHUNK = 16
N_CHUNKS = ROWS_PER_W // CHUNK
NBUF = 6


def _build_kernel(month_ref, hour_ref, comb_ref):
    m = month_ref[...].reshape(MONTH_ROWS, 1, D_MODEL)
    h = hour_ref[...].reshape(1, HOUR_ROWS, D_MODEL)
    comb_ref[...] = (m + h).reshape(COMB_ROWS, D_MODEL)


def _sc_kernel(cidx_hbm, comb_hbm, out_hbm, cidx_v, *bufs_and_sems):
    bufs = bufs_and_sems[:NBUF]
    gsems = bufs_and_sems[NBUF:2 * NBUF]
    osems = bufs_and_sems[2 * NBUF:]
    cid = lax.axis_index("c")
    sid = lax.axis_index("s")
    wid = sid * NC + cid
    base = wid * ROWS_PER_W

    pltpu.sync_copy(cidx_hbm.at[wid], cidx_v)

    gat_d = [None] * NBUF
    out_d = [None] * NBUF

    for c in range(NBUF):
        gat_d[c] = pltpu.async_copy(
            comb_hbm.at[cidx_v.at[c]], bufs[c], gsems[c])
    for c in range(N_CHUNKS):
        b = c % NBUF
        gat_d[b].wait()
        out_d[b] = pltpu.async_copy(
            bufs[b], out_hbm.at[pl.ds(base + c * CHUNK, CHUNK)], osems[b])
        if c + NBUF < N_CHUNKS:
            out_d[b].wait()
            gat_d[b] = pltpu.async_copy(
                comb_hbm.at[cidx_v.at[c + NBUF]], bufs[b], gsems[b])
    for c in range(N_CHUNKS - NBUF, N_CHUNKS):
        out_d[c % NBUF].wait()


@jax.jit
def _run(cidx, month_table, hour_table):
    comb = pl.pallas_call(
        _build_kernel,
        out_shape=jax.ShapeDtypeStruct((COMB_ROWS, D_MODEL), jnp.float32),
    )(month_table, hour_table)

    mesh = plsc.VectorSubcoreMesh(core_axis_name="c", subcore_axis_name="s")
    k = functools.partial(
        pl.kernel,
        out_type=jax.ShapeDtypeStruct((N_TOTAL, D_MODEL), jnp.float32),
        mesh=mesh,
        scratch_types=[
            pltpu.VMEM((N_CHUNKS, CHUNK), jnp.int32),
            *[pltpu.VMEM((CHUNK, D_MODEL), jnp.float32) for _ in range(NBUF)],
            *[pltpu.SemaphoreType.DMA for _ in range(2 * NBUF)],
        ],
    )(_sc_kernel)
    return k(cidx, comb)


def kernel(x, hour_table, month_table, minute_table):
    xi = x.astype(jnp.int32).reshape(N_TOTAL, 2)
    cidx = (xi[:, 0] * HOUR_ROWS + xi[:, 1]).reshape(NW, N_CHUNKS, CHUNK)
    out = _run(cidx, month_table, hour_table)
    return out.reshape(4, 4096, D_MODEL)
